# baseline (speedup 1.0000x reference)
import functools

import jax
import jax.numpy as jnp
from jax import lax
from jax.experimental import pallas as pl
from jax.experimental.pallas import tpu as pltpu
from jax.experimental.pallas import tpu_sc as plsc

_ROWS = 4 * 2048
_COLS = 1024
_CHUNK = 16
_NBUF = 4


def kernel(input1, output1):
    info = plsc.get_sparse_core_info()
    nc, ns = info.num_cores, info.num_subcores
    nw = nc * ns
    rows_per_w = _ROWS // nw
    nchunks = rows_per_w // _CHUNK

    mesh = plsc.VectorSubcoreMesh(core_axis_name="c", subcore_axis_name="s")

    @functools.partial(
        pl.kernel,
        mesh=mesh,
        out_type=jax.ShapeDtypeStruct((_ROWS, _COLS), jnp.float32),
        scratch_types=[
            pltpu.VMEM((_NBUF, _CHUNK, _COLS), jnp.float32),
            pltpu.SemaphoreType.DMA,
            pltpu.SemaphoreType.DMA,
        ],
    )
    def copy_k(in_hbm, out_hbm, buf, rsem, wsem):
        wid = lax.axis_index("s") * nc + lax.axis_index("c")
        base = wid * rows_per_w

        reads = []
        writes = []
        for i in range(min(_NBUF, nchunks)):
            reads.append(
                pltpu.async_copy(
                    in_hbm.at[pl.ds(base + i * _CHUNK, _CHUNK)],
                    buf.at[i % _NBUF],
                    rsem,
                )
            )
        for i in range(nchunks):
            reads[i].wait()
            writes.append(
                pltpu.async_copy(
                    buf.at[i % _NBUF],
                    out_hbm.at[pl.ds(base + i * _CHUNK, _CHUNK)],
                    wsem,
                )
            )
            nxt = i + _NBUF
            if nxt < nchunks:
                writes[i].wait()
                reads.append(
                    pltpu.async_copy(
                        in_hbm.at[pl.ds(base + nxt * _CHUNK, _CHUNK)],
                        buf.at[nxt % _NBUF],
                        rsem,
                    )
                )
        for w in writes[-min(_NBUF, nchunks):]:
            w.wait()

    out = copy_k(input1.reshape(_ROWS, _COLS))
    return out.reshape(input1.shape).astype(output1.dtype)


# speedup vs baseline: 1.0091x; 1.0091x over previous
import functools

import jax
import jax.numpy as jnp
from jax import lax
from jax.experimental import pallas as pl
from jax.experimental.pallas import tpu as pltpu
from jax.experimental.pallas import tpu_sc as plsc

_ROWS = 4 * 2048
_COLS = 1024
_CHUNK = 32
_NBUF = 2


def kernel(input1, output1):
    info = plsc.get_sparse_core_info()
    nc, ns = info.num_cores, info.num_subcores
    nw = nc * ns
    rows_per_w = _ROWS // nw
    nchunks = rows_per_w // _CHUNK

    mesh = plsc.VectorSubcoreMesh(core_axis_name="c", subcore_axis_name="s")

    @functools.partial(
        pl.kernel,
        mesh=mesh,
        out_type=jax.ShapeDtypeStruct((_ROWS, _COLS), jnp.float32),
        scratch_types=[
            pltpu.VMEM((_NBUF, _CHUNK, _COLS), jnp.float32),
            pltpu.SemaphoreType.DMA,
            pltpu.SemaphoreType.DMA,
        ],
    )
    def copy_k(in_hbm, out_hbm, buf, rsem, wsem):
        wid = lax.axis_index("s") * nc + lax.axis_index("c")
        base = wid * rows_per_w

        reads = []
        writes = []
        for i in range(min(_NBUF, nchunks)):
            reads.append(
                pltpu.async_copy(
                    in_hbm.at[pl.ds(base + i * _CHUNK, _CHUNK)],
                    buf.at[i % _NBUF],
                    rsem,
                )
            )
        for i in range(nchunks):
            reads[i].wait()
            writes.append(
                pltpu.async_copy(
                    buf.at[i % _NBUF],
                    out_hbm.at[pl.ds(base + i * _CHUNK, _CHUNK)],
                    wsem,
                )
            )
            nxt = i + _NBUF
            if nxt < nchunks:
                writes[i].wait()
                reads.append(
                    pltpu.async_copy(
                        in_hbm.at[pl.ds(base + nxt * _CHUNK, _CHUNK)],
                        buf.at[nxt % _NBUF],
                        rsem,
                    )
                )
        for w in writes[-min(_NBUF, nchunks):]:
            w.wait()

    out = copy_k(input1.reshape(_ROWS, _COLS))
    return out.reshape(input1.shape).astype(output1.dtype)


# speedup vs baseline: 1.0102x; 1.0012x over previous
import functools

import jax
import jax.numpy as jnp
from jax import lax
from jax.experimental import pallas as pl
from jax.experimental.pallas import tpu as pltpu
from jax.experimental.pallas import tpu_sc as plsc

_ROWS = 4 * 2048
_COLS = 1024
_CHUNK = 32
_NBUF = 2


def kernel(input1, output1):
    info = plsc.get_sparse_core_info()
    nc, ns = info.num_cores, info.num_subcores
    nw = nc * ns
    rows_per_w = _ROWS // nw
    nchunks = rows_per_w // _CHUNK

    mesh = plsc.VectorSubcoreMesh(core_axis_name="c", subcore_axis_name="s")

    @functools.partial(
        pl.kernel,
        mesh=mesh,
        out_type=jax.ShapeDtypeStruct((_ROWS, _COLS), jnp.float32),
        scratch_types=[
            pltpu.VMEM((_NBUF, _CHUNK, _COLS), jnp.float32),
            pltpu.SemaphoreType.DMA,
            pltpu.SemaphoreType.DMA,
        ],
    )
    def copy_k(in_hbm, out_hbm, buf, rsem, wsem):
        wid = lax.axis_index("c") * ns + lax.axis_index("s")
        base = wid * rows_per_w

        reads = []
        writes = []
        for i in range(min(_NBUF, nchunks)):
            reads.append(
                pltpu.async_copy(
                    in_hbm.at[pl.ds(base + i * _CHUNK, _CHUNK)],
                    buf.at[i % _NBUF],
                    rsem,
                )
            )
        for i in range(nchunks):
            reads[i].wait()
            writes.append(
                pltpu.async_copy(
                    buf.at[i % _NBUF],
                    out_hbm.at[pl.ds(base + i * _CHUNK, _CHUNK)],
                    wsem,
                )
            )
            nxt = i + _NBUF
            if nxt < nchunks:
                writes[i].wait()
                reads.append(
                    pltpu.async_copy(
                        in_hbm.at[pl.ds(base + nxt * _CHUNK, _CHUNK)],
                        buf.at[nxt % _NBUF],
                        rsem,
                    )
                )
        for w in writes[-min(_NBUF, nchunks):]:
            w.wait()

    out = copy_k(input1.reshape(_ROWS, _COLS))
    return out.reshape(input1.shape).astype(output1.dtype)
